# baseline (speedup 1.0000x reference)
import functools

import jax
import jax.numpy as jnp
from jax import lax
from jax.experimental import pallas as pl
from jax.experimental.pallas import tpu as pltpu
from jax.experimental.pallas import tpu_sc as plsc

N = 10000
E = 320000
H = 128
EPS = 1e-3

NC = 2
NS = 16
NW = NC * NS
B = 128
K = 80
E_PAD = NW * K * B
N_PAD = 10112
RPS = N_PAD // NS

_mesh = plsc.VectorSubcoreMesh(core_axis_name="c", subcore_axis_name="s")


@functools.partial(
    pl.kernel,
    out_type=jax.ShapeDtypeStruct((NC, N_PAD, H), jnp.float32),
    mesh=_mesh,
    scratch_types=[
        pltpu.VMEM((K, B), jnp.int32),
        pltpu.VMEM((B, H), jnp.float32),
        pltpu.VMEM_SHARED((N_PAD, H), jnp.float32),
        pltpu.SemaphoreType.DMA,
    ],
)
def _sc_degree_hist(dstw_hbm, ones_hbm, zeros_hbm, out_hbm,
                    dst_v, ones_v, hist_sh, hsem):
    cid = lax.axis_index("c")
    sid = lax.axis_index("s")
    wid = sid * NC + cid
    pltpu.sync_copy(zeros_hbm.at[pl.ds(sid * RPS, RPS)],
                    hist_sh.at[pl.ds(sid * RPS, RPS)])
    pltpu.sync_copy(dstw_hbm.at[wid], dst_v)
    pltpu.sync_copy(ones_hbm, ones_v)
    plsc.subcore_barrier()

    @pl.loop(0, K, step=8)
    def _(o):
        for b in range(8):
            pltpu.async_copy(ones_v, hist_sh.at[dst_v.at[o + b]], hsem,
                             add=True)
        for b in range(8):
            pltpu.make_async_copy(ones_v, hist_sh.at[dst_v.at[o + b]],
                                  hsem).wait()

    plsc.subcore_barrier()
    pltpu.sync_copy(hist_sh.at[pl.ds(sid * RPS, RPS)],
                    out_hbm.at[cid, pl.ds(sid * RPS, RPS)])


NPH = 2
KP = K // NPH


@functools.partial(
    pl.kernel,
    out_type=jax.ShapeDtypeStruct((NC, N_PAD, H), jnp.float32),
    mesh=_mesh,
    scratch_types=[
        pltpu.VMEM((KP, B), jnp.int32),
        pltpu.VMEM((KP, B), jnp.int32),
    ] + [pltpu.VMEM((B, H), jnp.float32)] * 2
      + [pltpu.SemaphoreType.DMA] * 4
      + [pltpu.VMEM_SHARED((N_PAD, H), jnp.float32)],
)
def _sc_scatter_add(hs_hbm, srcw_hbm, dstw_hbm, zeros_hbm, out_hbm,
                    src_v, dst_v, buf0, buf1, gs0, gs1, ws0, ws1, agg_sh):
    bufs = (buf0, buf1)
    gsems = (gs0, gs1)
    wsems = (ws0, ws1)
    cid = lax.axis_index("c")
    sid = lax.axis_index("s")
    wid = sid * NC + cid
    pltpu.sync_copy(zeros_hbm.at[pl.ds(sid * RPS, RPS)],
                    agg_sh.at[pl.ds(sid * RPS, RPS)])
    plsc.subcore_barrier()

    def g_copy(j, r):
        return pltpu.make_async_copy(hs_hbm.at[src_v.at[j]], bufs[r], gsems[r])

    def s_desc(j, r):
        return pltpu.make_async_copy(bufs[r], agg_sh.at[dst_v.at[j]], wsems[r])

    for p in range(NPH):
        pltpu.sync_copy(srcw_hbm.at[wid, pl.ds(p * KP, KP)], src_v)
        pltpu.sync_copy(dstw_hbm.at[wid, pl.ds(p * KP, KP)], dst_v)
        for r in range(2):
            g_copy(r, r).start()

        @pl.loop(0, KP, step=2)
        def _(o):
            for r in range(2):
                g_copy(o + r, r).wait()
                pltpu.async_copy(bufs[r], agg_sh.at[dst_v.at[o + r]],
                                 wsems[r], add=True)

            @pl.when(o + 2 < KP)
            def _():
                for r in range(2):
                    s_desc(o + r, r).wait()
                    g_copy(o + 2 + r, r).start()

        for r in range(2):
            s_desc(KP - 2 + r, r).wait()

    plsc.subcore_barrier()
    pltpu.sync_copy(agg_sh.at[pl.ds(sid * RPS, RPS)],
                    out_hbm.at[cid, pl.ds(sid * RPS, RPS)])


def _make_score_gather(k_chunks):
    e_half = NW * k_chunks * B

    @functools.partial(
        pl.kernel,
        out_type=tuple(jax.ShapeDtypeStruct((e_half, H), jnp.float32)
                       for _ in range(4)),
        mesh=_mesh,
        scratch_types=[pltpu.VMEM((k_chunks, B), jnp.int32)] * 4
                     + [pltpu.VMEM((B, H), jnp.float32)] * 4
                     + [pltpu.SemaphoreType.DMA] * 8,
    )
    def score_gather(hw_hbm, h_hbm, srcw_hbm, dstw_hbm, n0w_hbm, n1w_hbm,
                     s_out, d_out, n0_out, n1_out,
                     src_v, dst_v, n0_v, n1_v, *scratch):
        bufs = scratch[:4]
        gsems = scratch[4:8]
        wsems = scratch[8:12]
        cid = lax.axis_index("c")
        sid = lax.axis_index("s")
        wid = sid * NC + cid
        pltpu.sync_copy(srcw_hbm.at[wid], src_v)
        pltpu.sync_copy(dstw_hbm.at[wid], dst_v)
        pltpu.sync_copy(n0w_hbm.at[wid], n0_v)
        pltpu.sync_copy(n1w_hbm.at[wid], n1_v)
        base = wid * k_chunks * B

        tbls = (hw_hbm, h_hbm, h_hbm, h_hbm)
        idxs = (src_v, dst_v, n0_v, n1_v)
        outs = (s_out, d_out, n0_out, n1_out)

        def g_copy(j, b):
            return pltpu.make_async_copy(tbls[b].at[idxs[b].at[j]], bufs[b],
                                         gsems[b])

        def w_copy(j, b):
            return pltpu.make_async_copy(bufs[b],
                                         outs[b].at[pl.ds(base + j * B, B)],
                                         wsems[b])

        for b in range(4):
            g_copy(0, b).start()

        @pl.loop(0, k_chunks)
        def _(j):
            for b in range(4):
                g_copy(j, b).wait()
                w_copy(j, b).start()

            @pl.when(j + 1 < k_chunks)
            def _():
                for b in range(4):
                    w_copy(j, b).wait()
                    g_copy(j + 1, b).start()

        for b in range(4):
            w_copy(k_chunks - 1, b).wait()

    return score_gather


KH = K // 2
EH = NW * KH * B
_sc_score_gather = _make_score_gather(KH)


def _mm(a, b):
    return jnp.dot(a, b, preferred_element_type=jnp.float32)


def _softplus(x):
    return jnp.maximum(x, 0.0) + jnp.log(1.0 + jnp.exp(-jnp.abs(x)))


def _tc_dense0_body(x_ref, wemb_ref, bemb_ref, w1s_ref, w1d_ref, b1_ref,
                    h0_ref, hs1_ref, hd1_ref):
    h0 = _mm(x_ref[...], wemb_ref[...]) + bemb_ref[...]
    h0_ref[...] = h0
    hs1_ref[...] = _mm(h0, w1s_ref[...])
    hd1_ref[...] = _mm(h0, w1d_ref[...]) + b1_ref[...]


def _node_update(aggp_ref, histp_ref, hd_ref, hprev_ref, scale_ref, bias_ref):
    agg = aggp_ref[0, :N, :] + aggp_ref[1, :N, :]
    deg = histp_ref[0, :N, 0:1] + histp_ref[1, :N, 0:1]
    agg = agg + deg * hd_ref[...]
    t = 1.0 / (1.0 + jnp.exp(-agg)) + _softplus(hprev_ref[...])
    return jnp.maximum(t * scale_ref[...] + bias_ref[...], 0.0)


def _tc_dense_mid_body(aggp_ref, histp_ref, hd_ref, hprev_ref, scale_ref,
                       bias_ref, wns_ref, wnd_ref, bn_ref,
                       h_ref, hsn_ref, hdn_ref):
    h = _node_update(aggp_ref, histp_ref, hd_ref, hprev_ref, scale_ref, bias_ref)
    h_ref[...] = h
    hsn_ref[...] = _mm(h, wns_ref[...])
    hdn_ref[...] = _mm(h, wnd_ref[...]) + bn_ref[...]


def _tc_dense_fin_body(aggp_ref, histp_ref, hd_ref, hprev_ref, scale_ref,
                       bias_ref, wrel_ref, h_ref, hw_ref):
    h = _node_update(aggp_ref, histp_ref, hd_ref, hprev_ref, scale_ref, bias_ref)
    h_ref[:N, :] = h
    h_ref[N:, :] = jnp.zeros((N_PAD - N, H), jnp.float32)
    hw_ref[:N, :] = h * wrel_ref[...]
    hw_ref[N:, :] = jnp.zeros((N_PAD - N, H), jnp.float32)


SB = 2048


def _tc_score_body(s_ref, d_ref, n0_ref, n1_ref, pos_ref, neg0_ref, neg1_ref):
    s = s_ref[...]
    pos_ref[...] = jnp.sum(s * d_ref[...], axis=1).reshape(SB // 128, 128)
    neg0_ref[...] = jnp.sum(s * n0_ref[...], axis=1).reshape(SB // 128, 128)
    neg1_ref[...] = jnp.sum(s * n1_ref[...], axis=1).reshape(SB // 128, 128)


def _nh(n_out):
    return tuple(jax.ShapeDtypeStruct((N, H), jnp.float32) for _ in range(n_out))


_tc_dense0 = pl.pallas_call(_tc_dense0_body, out_shape=_nh(3))
_tc_dense_mid = pl.pallas_call(_tc_dense_mid_body, out_shape=_nh(3))
_tc_dense_fin = pl.pallas_call(
    _tc_dense_fin_body,
    out_shape=tuple(jax.ShapeDtypeStruct((N_PAD, H), jnp.float32)
                    for _ in range(2)))

_tc_score = pl.pallas_call(
    _tc_score_body,
    grid=(EH // SB,),
    in_specs=[pl.BlockSpec((SB, H), lambda i: (i, 0)) for _ in range(4)],
    out_specs=[pl.BlockSpec((SB // 128, 128), lambda i: (i, 0)) for _ in range(3)],
    out_shape=tuple(jax.ShapeDtypeStruct((EH // 128, 128), jnp.float32)
                    for _ in range(3)),
)


def kernel(x, edge_index, neg_dst, W_emb, b_emb, conv_w, conv_b, bn_gamma,
           bn_beta, bn_mean, bn_var, w_relation):
    f32 = jnp.float32
    src = edge_index[0]
    dst = edge_index[1]
    pad_e = E_PAD - E
    src_p = jnp.concatenate([src, jnp.zeros((pad_e,), jnp.int32)]).reshape(NW, K, B)
    dst_p = jnp.concatenate([dst, jnp.full((pad_e,), N, jnp.int32)]).reshape(NW, K, B)
    nd0_p = jnp.concatenate([neg_dst[0::2], jnp.zeros((pad_e,), jnp.int32)]).reshape(NW, K, B)
    nd1_p = jnp.concatenate([neg_dst[1::2], jnp.zeros((pad_e,), jnp.int32)]).reshape(NW, K, B)

    zerosN = jnp.zeros((N_PAD, H), f32)
    ones_col = jnp.zeros((B, H), f32).at[:, 0].set(1.0)

    scale = bn_gamma / jnp.sqrt(bn_var + EPS)
    bias = bn_beta - bn_mean * scale
    w1s, w1d = conv_w[0, :H, :], conv_w[0, H:, :]
    w2s, w2d = conv_w[1, :H, :], conv_w[1, H:, :]

    hist_p = _sc_degree_hist(dst_p, ones_col, zerosN)
    h0, hs1, hd1 = _tc_dense0(x, W_emb, b_emb.reshape(1, H), w1s, w1d,
                              conv_b[0].reshape(1, H))
    agg_p1 = _sc_scatter_add(hs1, src_p, dst_p, zerosN)
    h1, hs2, hd2 = _tc_dense_mid(agg_p1, hist_p, hd1, h0,
                                 scale[0].reshape(1, H), bias[0].reshape(1, H),
                                 w2s, w2d, conv_b[1].reshape(1, H))
    agg_p2 = _sc_scatter_add(hs2, src_p, dst_p, zerosN)
    h2, hw = _tc_dense_fin(agg_p2, hist_p, hd2, h1,
                           scale[1].reshape(1, H), bias[1].reshape(1, H),
                           w_relation)
    halves = []
    for h in range(2):
        sl = slice(h * KH, (h + 1) * KH)
        rows = _sc_score_gather(hw, h2, src_p[:, sl], dst_p[:, sl],
                                nd0_p[:, sl], nd1_p[:, sl])
        halves.append(_tc_score(*rows))

    def _merge(a, b):
        stacked = jnp.concatenate([a.reshape(NW, KH, 128),
                                   b.reshape(NW, KH, 128)], axis=1)
        return stacked.reshape(-1)

    pos = _merge(halves[0][0], halves[1][0])[:E]
    n0 = _merge(halves[0][1], halves[1][1])[:E]
    n1 = _merge(halves[0][2], halves[1][2])[:E]
    neg = jnp.stack([n0, n1], axis=1).reshape(-1)
    return pos, neg


# speedup vs baseline: 1.0185x; 1.0185x over previous
import functools

import jax
import jax.numpy as jnp
from jax import lax
from jax.experimental import pallas as pl
from jax.experimental.pallas import tpu as pltpu
from jax.experimental.pallas import tpu_sc as plsc

N = 10000
E = 320000
H = 128
EPS = 1e-3

NC = 2
NS = 16
NW = NC * NS
B = 128
K = 80
E_PAD = NW * K * B
N_PAD = 10112
RPS = N_PAD // NS

_mesh = plsc.VectorSubcoreMesh(core_axis_name="c", subcore_axis_name="s")


@functools.partial(
    pl.kernel,
    out_type=jax.ShapeDtypeStruct((NC, N_PAD, H), jnp.float32),
    mesh=_mesh,
    scratch_types=[
        pltpu.VMEM((K, B), jnp.int32),
        pltpu.VMEM((B, H), jnp.float32),
        pltpu.VMEM_SHARED((N_PAD, H), jnp.float32),
        pltpu.SemaphoreType.DMA,
    ],
)
def _sc_degree_hist(dstw_hbm, ones_hbm, zeros_hbm, out_hbm,
                    dst_v, ones_v, hist_sh, hsem):
    cid = lax.axis_index("c")
    sid = lax.axis_index("s")
    wid = sid * NC + cid
    pltpu.sync_copy(zeros_hbm.at[pl.ds(sid * RPS, RPS)],
                    hist_sh.at[pl.ds(sid * RPS, RPS)])
    pltpu.sync_copy(dstw_hbm.at[wid], dst_v)
    pltpu.sync_copy(ones_hbm, ones_v)
    plsc.subcore_barrier()

    @pl.loop(0, K, step=8)
    def _(o):
        for b in range(8):
            pltpu.async_copy(ones_v, hist_sh.at[dst_v.at[o + b]], hsem,
                             add=True)
        for b in range(8):
            pltpu.make_async_copy(ones_v, hist_sh.at[dst_v.at[o + b]],
                                  hsem).wait()

    plsc.subcore_barrier()
    pltpu.sync_copy(hist_sh.at[pl.ds(sid * RPS, RPS)],
                    out_hbm.at[cid, pl.ds(sid * RPS, RPS)])


NPH = 2
KP = K // NPH


@functools.partial(
    pl.kernel,
    out_type=jax.ShapeDtypeStruct((NC, N_PAD, H), jnp.float32),
    mesh=_mesh,
    scratch_types=[
        pltpu.VMEM((KP, B), jnp.int32),
        pltpu.VMEM((KP, B), jnp.int32),
    ] + [pltpu.VMEM((B, H), jnp.float32)] * 2
      + [pltpu.SemaphoreType.DMA] * 4
      + [pltpu.VMEM_SHARED((N_PAD, H), jnp.float32)],
)
def _sc_scatter_add(hs_hbm, srcw_hbm, dstw_hbm, zeros_hbm, out_hbm,
                    src_v, dst_v, buf0, buf1, gs0, gs1, ws0, ws1, agg_sh):
    bufs = (buf0, buf1)
    gsems = (gs0, gs1)
    wsems = (ws0, ws1)
    cid = lax.axis_index("c")
    sid = lax.axis_index("s")
    wid = sid * NC + cid
    pltpu.sync_copy(zeros_hbm.at[pl.ds(sid * RPS, RPS)],
                    agg_sh.at[pl.ds(sid * RPS, RPS)])
    plsc.subcore_barrier()

    def g_copy(j, r):
        return pltpu.make_async_copy(hs_hbm.at[src_v.at[j]], bufs[r], gsems[r])

    def s_desc(j, r):
        return pltpu.make_async_copy(bufs[r], agg_sh.at[dst_v.at[j]], wsems[r])

    for p in range(NPH):
        pltpu.sync_copy(srcw_hbm.at[wid, pl.ds(p * KP, KP)], src_v)
        pltpu.sync_copy(dstw_hbm.at[wid, pl.ds(p * KP, KP)], dst_v)
        for r in range(2):
            g_copy(r, r).start()

        @pl.loop(0, KP, step=2)
        def _(o):
            for r in range(2):
                g_copy(o + r, r).wait()
                pltpu.async_copy(bufs[r], agg_sh.at[dst_v.at[o + r]],
                                 wsems[r], add=True)

            @pl.when(o + 2 < KP)
            def _():
                for r in range(2):
                    s_desc(o + r, r).wait()
                    g_copy(o + 2 + r, r).start()

        for r in range(2):
            s_desc(KP - 2 + r, r).wait()

    plsc.subcore_barrier()
    pltpu.sync_copy(agg_sh.at[pl.ds(sid * RPS, RPS)],
                    out_hbm.at[cid, pl.ds(sid * RPS, RPS)])


def _make_score_gather(k_chunks):
    e_half = NW * k_chunks * B

    @functools.partial(
        pl.kernel,
        out_type=tuple(jax.ShapeDtypeStruct((e_half, H), jnp.float32)
                       for _ in range(4)),
        mesh=_mesh,
        scratch_types=[pltpu.VMEM((k_chunks, B), jnp.int32)] * 4
                     + [pltpu.VMEM((B, H), jnp.float32)] * 4
                     + [pltpu.SemaphoreType.DMA] * 8,
    )
    def score_gather(hw_hbm, h_hbm, srcw_hbm, dstw_hbm, n0w_hbm, n1w_hbm,
                     s_out, d_out, n0_out, n1_out,
                     src_v, dst_v, n0_v, n1_v, *scratch):
        bufs = scratch[:4]
        gsems = scratch[4:8]
        wsems = scratch[8:12]
        cid = lax.axis_index("c")
        sid = lax.axis_index("s")
        wid = sid * NC + cid
        pltpu.sync_copy(srcw_hbm.at[wid], src_v)
        pltpu.sync_copy(dstw_hbm.at[wid], dst_v)
        pltpu.sync_copy(n0w_hbm.at[wid], n0_v)
        pltpu.sync_copy(n1w_hbm.at[wid], n1_v)
        base = wid * k_chunks * B

        tbls = (hw_hbm, h_hbm, h_hbm, h_hbm)
        idxs = (src_v, dst_v, n0_v, n1_v)
        outs = (s_out, d_out, n0_out, n1_out)

        def g_copy(j, b):
            return pltpu.make_async_copy(tbls[b].at[idxs[b].at[j]], bufs[b],
                                         gsems[b])

        def w_copy(j, b):
            return pltpu.make_async_copy(bufs[b],
                                         outs[b].at[pl.ds(base + j * B, B)],
                                         wsems[b])

        for b in range(4):
            g_copy(0, b).start()

        @pl.loop(0, k_chunks)
        def _(j):
            for b in range(4):
                g_copy(j, b).wait()
                w_copy(j, b).start()

            @pl.when(j + 1 < k_chunks)
            def _():
                for b in range(4):
                    w_copy(j, b).wait()
                    g_copy(j + 1, b).start()

        for b in range(4):
            w_copy(k_chunks - 1, b).wait()

    return score_gather


KH = K
EH = NW * KH * B
_sc_score_gather = _make_score_gather(KH)


def _mm(a, b):
    return jnp.dot(a, b, preferred_element_type=jnp.float32)


def _softplus(x):
    return jnp.maximum(x, 0.0) + jnp.log(1.0 + jnp.exp(-jnp.abs(x)))


def _tc_dense0_body(x_ref, wemb_ref, bemb_ref, w1s_ref, w1d_ref, b1_ref,
                    h0_ref, hs1_ref, hd1_ref):
    h0 = _mm(x_ref[...], wemb_ref[...]) + bemb_ref[...]
    h0_ref[...] = h0
    hs1_ref[...] = _mm(h0, w1s_ref[...])
    hd1_ref[...] = _mm(h0, w1d_ref[...]) + b1_ref[...]


def _node_update(aggp_ref, histp_ref, hd_ref, hprev_ref, scale_ref, bias_ref):
    agg = aggp_ref[0, :N, :] + aggp_ref[1, :N, :]
    deg = histp_ref[0, :N, 0:1] + histp_ref[1, :N, 0:1]
    agg = agg + deg * hd_ref[...]
    t = 1.0 / (1.0 + jnp.exp(-agg)) + _softplus(hprev_ref[...])
    return jnp.maximum(t * scale_ref[...] + bias_ref[...], 0.0)


def _tc_dense_mid_body(aggp_ref, histp_ref, hd_ref, hprev_ref, scale_ref,
                       bias_ref, wns_ref, wnd_ref, bn_ref,
                       h_ref, hsn_ref, hdn_ref):
    h = _node_update(aggp_ref, histp_ref, hd_ref, hprev_ref, scale_ref, bias_ref)
    h_ref[...] = h
    hsn_ref[...] = _mm(h, wns_ref[...])
    hdn_ref[...] = _mm(h, wnd_ref[...]) + bn_ref[...]


def _tc_dense_fin_body(aggp_ref, histp_ref, hd_ref, hprev_ref, scale_ref,
                       bias_ref, wrel_ref, h_ref, hw_ref):
    h = _node_update(aggp_ref, histp_ref, hd_ref, hprev_ref, scale_ref, bias_ref)
    h_ref[:N, :] = h
    h_ref[N:, :] = jnp.zeros((N_PAD - N, H), jnp.float32)
    hw_ref[:N, :] = h * wrel_ref[...]
    hw_ref[N:, :] = jnp.zeros((N_PAD - N, H), jnp.float32)


SB = 2048


def _tc_score_body(s_ref, d_ref, n0_ref, n1_ref, pos_ref, neg0_ref, neg1_ref):
    s = s_ref[...]
    pos_ref[...] = jnp.sum(s * d_ref[...], axis=1).reshape(SB // 128, 128)
    neg0_ref[...] = jnp.sum(s * n0_ref[...], axis=1).reshape(SB // 128, 128)
    neg1_ref[...] = jnp.sum(s * n1_ref[...], axis=1).reshape(SB // 128, 128)


def _nh(n_out):
    return tuple(jax.ShapeDtypeStruct((N, H), jnp.float32) for _ in range(n_out))


_tc_dense0 = pl.pallas_call(_tc_dense0_body, out_shape=_nh(3))
_tc_dense_mid = pl.pallas_call(_tc_dense_mid_body, out_shape=_nh(3))
_tc_dense_fin = pl.pallas_call(
    _tc_dense_fin_body,
    out_shape=tuple(jax.ShapeDtypeStruct((N_PAD, H), jnp.float32)
                    for _ in range(2)))

_tc_score = pl.pallas_call(
    _tc_score_body,
    grid=(EH // SB,),
    in_specs=[pl.BlockSpec((SB, H), lambda i: (i, 0)) for _ in range(4)],
    out_specs=[pl.BlockSpec((SB // 128, 128), lambda i: (i, 0)) for _ in range(3)],
    out_shape=tuple(jax.ShapeDtypeStruct((EH // 128, 128), jnp.float32)
                    for _ in range(3)),
)


def kernel(x, edge_index, neg_dst, W_emb, b_emb, conv_w, conv_b, bn_gamma,
           bn_beta, bn_mean, bn_var, w_relation):
    f32 = jnp.float32
    src = edge_index[0]
    dst = edge_index[1]
    pad_e = E_PAD - E
    src_p = jnp.concatenate([src, jnp.zeros((pad_e,), jnp.int32)]).reshape(NW, K, B)
    dst_p = jnp.concatenate([dst, jnp.full((pad_e,), N, jnp.int32)]).reshape(NW, K, B)
    nd0_p = jnp.concatenate([neg_dst[0::2], jnp.zeros((pad_e,), jnp.int32)]).reshape(NW, K, B)
    nd1_p = jnp.concatenate([neg_dst[1::2], jnp.zeros((pad_e,), jnp.int32)]).reshape(NW, K, B)

    zerosN = jnp.zeros((N_PAD, H), f32)
    ones_col = jnp.zeros((B, H), f32).at[:, 0].set(1.0)

    scale = bn_gamma / jnp.sqrt(bn_var + EPS)
    bias = bn_beta - bn_mean * scale
    w1s, w1d = conv_w[0, :H, :], conv_w[0, H:, :]
    w2s, w2d = conv_w[1, :H, :], conv_w[1, H:, :]

    hist_p = _sc_degree_hist(dst_p, ones_col, zerosN)
    h0, hs1, hd1 = _tc_dense0(x, W_emb, b_emb.reshape(1, H), w1s, w1d,
                              conv_b[0].reshape(1, H))
    agg_p1 = _sc_scatter_add(hs1, src_p, dst_p, zerosN)
    h1, hs2, hd2 = _tc_dense_mid(agg_p1, hist_p, hd1, h0,
                                 scale[0].reshape(1, H), bias[0].reshape(1, H),
                                 w2s, w2d, conv_b[1].reshape(1, H))
    agg_p2 = _sc_scatter_add(hs2, src_p, dst_p, zerosN)
    h2, hw = _tc_dense_fin(agg_p2, hist_p, hd2, h1,
                           scale[1].reshape(1, H), bias[1].reshape(1, H),
                           w_relation)
    rows = _sc_score_gather(hw, h2, src_p, dst_p, nd0_p, nd1_p)
    pos2d, neg0_2d, neg1_2d = _tc_score(*rows)

    pos = pos2d.reshape(-1)[:E]
    n0 = neg0_2d.reshape(-1)[:E]
    n1 = neg1_2d.reshape(-1)[:E]
    neg = jnp.stack([n0, n1], axis=1).reshape(-1)
    return pos, neg


# speedup vs baseline: 1.0662x; 1.0469x over previous
import dataclasses
import functools

import jax
import jax.numpy as jnp
from jax import lax
from jax.experimental import pallas as pl
from jax.experimental.pallas import tpu as pltpu
from jax.experimental.pallas import tpu_sc as plsc

N = 10000
E = 320000
H = 128
EPS = 1e-3

NC = 2
NS = 16
NW = NC * NS
B = 128
K = 80
E_PAD = NW * K * B
N_PAD = 10112
RPS = N_PAD // NS

_mesh = plsc.VectorSubcoreMesh(core_axis_name="c", subcore_axis_name="s")


_cp_no_layout = pltpu.CompilerParams()
if "needs_layout_passes" in pltpu.CompilerParams.__dataclass_fields__:
    _cp_no_layout = dataclasses.replace(_cp_no_layout, needs_layout_passes=False)


@functools.partial(
    pl.kernel,
    out_type=jax.ShapeDtypeStruct((NW, N_PAD), jnp.float32),
    mesh=_mesh,
    compiler_params=_cp_no_layout,
    scratch_types=[
        pltpu.VMEM((K, B), jnp.int32),
        pltpu.VMEM((N_PAD,), jnp.float32),
    ],
)
def _sc_degree_hist(dstw_hbm, zeros_hbm, out_hbm, dst_v, loc):
    cid = lax.axis_index("c")
    sid = lax.axis_index("s")
    wid = sid * NC + cid
    pltpu.sync_copy(zeros_hbm, loc)
    pltpu.sync_copy(dstw_hbm.at[wid], dst_v)
    ones = jnp.full((16,), 1.0, jnp.float32)

    @pl.loop(0, K)
    def _(j):
        @pl.loop(0, B, step=16)
        def _(g):
            idx = dst_v[j, pl.ds(g, 16)]
            plsc.addupdate_scatter(loc, [idx], ones)

    pltpu.sync_copy(loc, out_hbm.at[wid])


NPH = 2
KP = K // NPH


@functools.partial(
    pl.kernel,
    out_type=jax.ShapeDtypeStruct((NC, N_PAD, H), jnp.float32),
    mesh=_mesh,
    scratch_types=[
        pltpu.VMEM((KP, B), jnp.int32),
        pltpu.VMEM((KP, B), jnp.int32),
    ] + [pltpu.VMEM((B, H), jnp.float32)] * 2
      + [pltpu.SemaphoreType.DMA] * 4
      + [pltpu.VMEM_SHARED((N_PAD, H), jnp.float32)],
)
def _sc_scatter_add(hs_hbm, srcw_hbm, dstw_hbm, zeros_hbm, out_hbm,
                    src_v, dst_v, buf0, buf1, gs0, gs1, ws0, ws1, agg_sh):
    bufs = (buf0, buf1)
    gsems = (gs0, gs1)
    wsems = (ws0, ws1)
    cid = lax.axis_index("c")
    sid = lax.axis_index("s")
    wid = sid * NC + cid
    pltpu.sync_copy(zeros_hbm.at[pl.ds(sid * RPS, RPS)],
                    agg_sh.at[pl.ds(sid * RPS, RPS)])
    plsc.subcore_barrier()

    def g_copy(j, r):
        return pltpu.make_async_copy(hs_hbm.at[src_v.at[j]], bufs[r], gsems[r])

    def s_desc(j, r):
        return pltpu.make_async_copy(bufs[r], agg_sh.at[dst_v.at[j]], wsems[r])

    for p in range(NPH):
        pltpu.sync_copy(srcw_hbm.at[wid, pl.ds(p * KP, KP)], src_v)
        pltpu.sync_copy(dstw_hbm.at[wid, pl.ds(p * KP, KP)], dst_v)
        for r in range(2):
            g_copy(r, r).start()

        @pl.loop(0, KP, step=2)
        def _(o):
            for r in range(2):
                g_copy(o + r, r).wait()
                pltpu.async_copy(bufs[r], agg_sh.at[dst_v.at[o + r]],
                                 wsems[r], add=True)

            @pl.when(o + 2 < KP)
            def _():
                for r in range(2):
                    s_desc(o + r, r).wait()
                    g_copy(o + 2 + r, r).start()

        for r in range(2):
            s_desc(KP - 2 + r, r).wait()

    plsc.subcore_barrier()
    pltpu.sync_copy(agg_sh.at[pl.ds(sid * RPS, RPS)],
                    out_hbm.at[cid, pl.ds(sid * RPS, RPS)])


def _make_score_gather(k_chunks):
    e_half = NW * k_chunks * B

    @functools.partial(
        pl.kernel,
        out_type=tuple(jax.ShapeDtypeStruct((e_half, H), jnp.float32)
                       for _ in range(4)),
        mesh=_mesh,
        scratch_types=[pltpu.VMEM((k_chunks, B), jnp.int32)] * 4
                     + [pltpu.VMEM((B, H), jnp.float32)] * 4
                     + [pltpu.SemaphoreType.DMA] * 8,
    )
    def score_gather(hw_hbm, h_hbm, srcw_hbm, dstw_hbm, n0w_hbm, n1w_hbm,
                     s_out, d_out, n0_out, n1_out,
                     src_v, dst_v, n0_v, n1_v, *scratch):
        bufs = scratch[:4]
        gsems = scratch[4:8]
        wsems = scratch[8:12]
        cid = lax.axis_index("c")
        sid = lax.axis_index("s")
        wid = sid * NC + cid
        pltpu.sync_copy(srcw_hbm.at[wid], src_v)
        pltpu.sync_copy(dstw_hbm.at[wid], dst_v)
        pltpu.sync_copy(n0w_hbm.at[wid], n0_v)
        pltpu.sync_copy(n1w_hbm.at[wid], n1_v)
        base = wid * k_chunks * B

        tbls = (hw_hbm, h_hbm, h_hbm, h_hbm)
        idxs = (src_v, dst_v, n0_v, n1_v)
        outs = (s_out, d_out, n0_out, n1_out)

        def g_copy(j, b):
            return pltpu.make_async_copy(tbls[b].at[idxs[b].at[j]], bufs[b],
                                         gsems[b])

        def w_copy(j, b):
            return pltpu.make_async_copy(bufs[b],
                                         outs[b].at[pl.ds(base + j * B, B)],
                                         wsems[b])

        for b in range(4):
            g_copy(0, b).start()

        @pl.loop(0, k_chunks)
        def _(j):
            for b in range(4):
                g_copy(j, b).wait()
                w_copy(j, b).start()

            @pl.when(j + 1 < k_chunks)
            def _():
                for b in range(4):
                    w_copy(j, b).wait()
                    g_copy(j + 1, b).start()

        for b in range(4):
            w_copy(k_chunks - 1, b).wait()

    return score_gather


KH = K
EH = NW * KH * B
_sc_score_gather = _make_score_gather(KH)


def _mm(a, b):
    return jnp.dot(a, b, preferred_element_type=jnp.float32)


def _softplus(x):
    return jnp.maximum(x, 0.0) + jnp.log(1.0 + jnp.exp(-jnp.abs(x)))


def _tc_dense0_body(x_ref, wemb_ref, bemb_ref, w1s_ref, w1d_ref, b1_ref,
                    h0_ref, hs1_ref, hd1_ref):
    h0 = _mm(x_ref[...], wemb_ref[...]) + bemb_ref[...]
    h0_ref[...] = h0
    hs1_ref[...] = _mm(h0, w1s_ref[...])
    hd1_ref[...] = _mm(h0, w1d_ref[...]) + b1_ref[...]


def _node_update(aggp_ref, histp_ref, hd_ref, hprev_ref, scale_ref, bias_ref):
    agg = aggp_ref[0, :N, :] + aggp_ref[1, :N, :]
    deg = lax.dot_general(histp_ref[...], jnp.ones((NW, 1), jnp.float32),
                          (((0,), (0,)), ((), ())),
                          preferred_element_type=jnp.float32)
    agg = agg + deg[:N, :] * hd_ref[...]
    t = 1.0 / (1.0 + jnp.exp(-agg)) + _softplus(hprev_ref[...])
    return jnp.maximum(t * scale_ref[...] + bias_ref[...], 0.0)


def _tc_dense_mid_body(aggp_ref, histp_ref, hd_ref, hprev_ref, scale_ref,
                       bias_ref, wns_ref, wnd_ref, bn_ref,
                       h_ref, hsn_ref, hdn_ref):
    h = _node_update(aggp_ref, histp_ref, hd_ref, hprev_ref, scale_ref, bias_ref)
    h_ref[...] = h
    hsn_ref[...] = _mm(h, wns_ref[...])
    hdn_ref[...] = _mm(h, wnd_ref[...]) + bn_ref[...]


def _tc_dense_fin_body(aggp_ref, histp_ref, hd_ref, hprev_ref, scale_ref,
                       bias_ref, wrel_ref, h_ref, hw_ref):
    h = _node_update(aggp_ref, histp_ref, hd_ref, hprev_ref, scale_ref, bias_ref)
    h_ref[:N, :] = h
    h_ref[N:, :] = jnp.zeros((N_PAD - N, H), jnp.float32)
    hw_ref[:N, :] = h * wrel_ref[...]
    hw_ref[N:, :] = jnp.zeros((N_PAD - N, H), jnp.float32)


SB = 2048


def _tc_score_body(s_ref, d_ref, n0_ref, n1_ref, pos_ref, neg0_ref, neg1_ref):
    s = s_ref[...]
    pos_ref[...] = jnp.sum(s * d_ref[...], axis=1).reshape(SB // 128, 128)
    neg0_ref[...] = jnp.sum(s * n0_ref[...], axis=1).reshape(SB // 128, 128)
    neg1_ref[...] = jnp.sum(s * n1_ref[...], axis=1).reshape(SB // 128, 128)


def _nh(n_out):
    return tuple(jax.ShapeDtypeStruct((N, H), jnp.float32) for _ in range(n_out))


_tc_dense0 = pl.pallas_call(_tc_dense0_body, out_shape=_nh(3))
_tc_dense_mid = pl.pallas_call(_tc_dense_mid_body, out_shape=_nh(3))
_tc_dense_fin = pl.pallas_call(
    _tc_dense_fin_body,
    out_shape=tuple(jax.ShapeDtypeStruct((N_PAD, H), jnp.float32)
                    for _ in range(2)))

_tc_score = pl.pallas_call(
    _tc_score_body,
    grid=(EH // SB,),
    in_specs=[pl.BlockSpec((SB, H), lambda i: (i, 0)) for _ in range(4)],
    out_specs=[pl.BlockSpec((SB // 128, 128), lambda i: (i, 0)) for _ in range(3)],
    out_shape=tuple(jax.ShapeDtypeStruct((EH // 128, 128), jnp.float32)
                    for _ in range(3)),
)


def kernel(x, edge_index, neg_dst, W_emb, b_emb, conv_w, conv_b, bn_gamma,
           bn_beta, bn_mean, bn_var, w_relation):
    f32 = jnp.float32
    src = edge_index[0]
    dst = edge_index[1]
    pad_e = E_PAD - E
    src_p = jnp.concatenate([src, jnp.zeros((pad_e,), jnp.int32)]).reshape(NW, K, B)
    dst_p = jnp.concatenate([dst, jnp.full((pad_e,), N, jnp.int32)]).reshape(NW, K, B)
    nd0_p = jnp.concatenate([neg_dst[0::2], jnp.zeros((pad_e,), jnp.int32)]).reshape(NW, K, B)
    nd1_p = jnp.concatenate([neg_dst[1::2], jnp.zeros((pad_e,), jnp.int32)]).reshape(NW, K, B)

    zerosN = jnp.zeros((N_PAD, H), f32)
    zeros1 = jnp.zeros((N_PAD,), f32)

    scale = bn_gamma / jnp.sqrt(bn_var + EPS)
    bias = bn_beta - bn_mean * scale
    w1s, w1d = conv_w[0, :H, :], conv_w[0, H:, :]
    w2s, w2d = conv_w[1, :H, :], conv_w[1, H:, :]

    hist_p = _sc_degree_hist(dst_p, zeros1)
    h0, hs1, hd1 = _tc_dense0(x, W_emb, b_emb.reshape(1, H), w1s, w1d,
                              conv_b[0].reshape(1, H))
    agg_p1 = _sc_scatter_add(hs1, src_p, dst_p, zerosN)
    h1, hs2, hd2 = _tc_dense_mid(agg_p1, hist_p, hd1, h0,
                                 scale[0].reshape(1, H), bias[0].reshape(1, H),
                                 w2s, w2d, conv_b[1].reshape(1, H))
    agg_p2 = _sc_scatter_add(hs2, src_p, dst_p, zerosN)
    h2, hw = _tc_dense_fin(agg_p2, hist_p, hd2, h1,
                           scale[1].reshape(1, H), bias[1].reshape(1, H),
                           w_relation)
    rows = _sc_score_gather(hw, h2, src_p, dst_p, nd0_p, nd1_p)
    pos2d, neg0_2d, neg1_2d = _tc_score(*rows)

    pos = pos2d.reshape(-1)[:E]
    n0 = neg0_2d.reshape(-1)[:E]
    n1 = neg1_2d.reshape(-1)[:E]
    neg = jnp.stack([n0, n1], axis=1).reshape(-1)
    return pos, neg
